# baseline (speedup 1.0000x reference)
import functools

import jax
import jax.numpy as jnp
import numpy as np
from jax.experimental import pallas as pl

BS = 50
N_NODE = 200
N = BS * N_NODE
E = 160000
HID = 200
SENT = 1024
SEQ = 100
N_ETYPE = 38
N_NTYPE = 4
HEADS = 4
DPH = HID // HEADS
K_LAYERS = 5
EP_1 = 1.0
EP_2 = 1.0
E_TOT = E + N


def _bn(x, g, b, m, v):
    return (x - m) / jnp.sqrt(v + 1e-5) * g + b


_BM = 1000


def _proj_body(x_ref, e_ref, w_ref, b_ref, o_ref):
    xx = jnp.concatenate([x_ref[...], e_ref[...]], axis=1)
    o_ref[...] = (
        jnp.dot(xx, w_ref[...], preferred_element_type=jnp.float32) + b_ref[...][None, :]
    )


def _node_proj(x, extra, w, b):
    return pl.pallas_call(
        _proj_body,
        grid=(N // _BM,),
        in_specs=[
            pl.BlockSpec((_BM, HID), lambda i: (i, 0)),
            pl.BlockSpec((_BM, HID), lambda i: (i, 0)),
            pl.BlockSpec((2 * HID, 3 * HID), lambda i: (0, 0)),
            pl.BlockSpec((3 * HID,), lambda i: (0,)),
        ],
        out_specs=pl.BlockSpec((_BM, 3 * HID), lambda i: (i, 0)),
        out_shape=jax.ShapeDtypeStruct((N, 3 * HID), jnp.float32),
    )(x, extra, w, b)


NCODE = 624


def _ttab_body(kx_ref, q_ref, keb_ref, ind_ref, o_ref):
    q = q_ref[...] * np.float32(1.0 / np.sqrt(DPH))
    a = jnp.dot(q * kx_ref[...], ind_ref[...], preferred_element_type=jnp.float32)
    qa = jnp.concatenate([q, a], axis=1)
    o_ref[...] = jnp.dot(qa, keb_ref[...], preferred_element_type=jnp.float32)


def _ttab(kx, qx, keb_t, ind):
    return pl.pallas_call(
        _ttab_body,
        grid=(N // _BM,),
        in_specs=[
            pl.BlockSpec((_BM, HID), lambda i: (i, 0)),
            pl.BlockSpec((_BM, HID), lambda i: (i, 0)),
            pl.BlockSpec((HID + HEADS, NCODE * HEADS), lambda i: (0, 0)),
            pl.BlockSpec((HID, HEADS), lambda i: (0, 0)),
        ],
        out_specs=pl.BlockSpec((_BM, NCODE * HEADS), lambda i: (i, 0)),
        out_shape=jax.ShapeDtypeStruct((N, NCODE * HEADS), jnp.float32),
    )(kx, qx, keb_t, ind)


def _mlp_body(aggr_ref, w1_ref, b1_ref, bn_ref, w2_ref, b2_ref, o_ref, mx_ref):
    h = (
        jnp.dot(aggr_ref[...], w1_ref[...], preferred_element_type=jnp.float32)
        + b1_ref[...][None, :]
    )
    bn = bn_ref[...]
    h = _bn(h, bn[0][None, :], bn[1][None, :], bn[2][None, :], bn[3][None, :])
    h = jnp.maximum(h, 0.0)
    out = (
        jnp.dot(h, w2_ref[...], preferred_element_type=jnp.float32) + b2_ref[...][None, :]
    )
    o_ref[...] = out
    bmax = jnp.max(jnp.sum(out * out, axis=1)).reshape(1, 1)

    @pl.when(pl.program_id(0) == 0)
    def _init():
        mx_ref[...] = bmax

    @pl.when(pl.program_id(0) != 0)
    def _acc():
        mx_ref[...] = jnp.maximum(mx_ref[...], bmax)


def _mlp_stage(aggr, w1t, b1, bn4, w2t, b2):
    return pl.pallas_call(
        _mlp_body,
        grid=(N // _BM,),
        in_specs=[
            pl.BlockSpec((_BM, HID), lambda i: (i, 0)),
            pl.BlockSpec((HID, HID), lambda i: (0, 0)),
            pl.BlockSpec((HID,), lambda i: (0,)),
            pl.BlockSpec((4, HID), lambda i: (0, 0)),
            pl.BlockSpec((HID, HID), lambda i: (0, 0)),
            pl.BlockSpec((HID,), lambda i: (0,)),
        ],
        out_specs=[
            pl.BlockSpec((_BM, HID), lambda i: (i, 0)),
            pl.BlockSpec((1, 1), lambda i: (0, 0)),
        ],
        out_shape=[
            jax.ShapeDtypeStruct((N, HID), jnp.float32),
            jax.ShapeDtypeStruct((1, 1), jnp.float32),
        ],
    )(aggr, w1t, b1, bn4, w2t, b2)


def _xattn_body(out_ref, wq_ref, bq_ref, kt_ref, v_ref, noise_ref, mx_ref, o_ref):
    out = out_ref[...]
    scale = 2.0 * jnp.sqrt(mx_ref[...]) / EP_2 / (EP_1 / 4.0)
    x = out * 0.5 * (1.0 + jax.lax.erf(out / np.float32(np.sqrt(2.0))))
    q = jnp.dot(x, wq_ref[...], preferred_element_type=jnp.float32) + bq_ref[...][None, :]
    qn = jnp.sqrt(jnp.sum(q * q, axis=1, keepdims=True))
    q = q / jnp.maximum(qn, 1e-12)
    kt = kt_ref[...].reshape(1, HID)
    att = jnp.sum(q * kt, axis=1, keepdims=True) / np.float32(np.sqrt(HID))
    att = att + noise_ref[...].reshape(N_NODE, 1) * scale
    o_ref[...] = att * v_ref[...].reshape(1, HID)


def _xattn_stage(out, wqt, bq, kt, v, noise, mx):
    return pl.pallas_call(
        _xattn_body,
        grid=(BS,),
        in_specs=[
            pl.BlockSpec((N_NODE, HID), lambda i: (i, 0)),
            pl.BlockSpec((HID, HID), lambda i: (0, 0)),
            pl.BlockSpec((HID,), lambda i: (0,)),
            pl.BlockSpec((1, 1, HID), lambda i: (i, 0, 0)),
            pl.BlockSpec((1, 1, HID), lambda i: (i, 0, 0)),
            pl.BlockSpec((1, 1, N_NODE), lambda i: (i, 0, 0)),
            pl.BlockSpec((1, 1), lambda i: (0, 0)),
        ],
        out_specs=pl.BlockSpec((N_NODE, HID), lambda i: (i, 0)),
        out_shape=jax.ShapeDtypeStruct((N, HID), jnp.float32),
    )(out, wqt, bq, kt.reshape(BS, 1, HID), v.reshape(BS, 1, HID),
      noise.reshape(BS, 1, N_NODE), mx)


def _edge_table(pe):
    codes = jnp.arange(624, dtype=jnp.int32)
    et = codes // 16
    ht = (codes // 4) % 4
    tt = codes % 4
    feats = jnp.concatenate(
        [
            jax.nn.one_hot(et, N_ETYPE + 1, dtype=jnp.float32),
            jax.nn.one_hot(ht, N_NTYPE, dtype=jnp.float32),
            jax.nn.one_hot(tt, N_NTYPE, dtype=jnp.float32),
        ],
        axis=1,
    )
    h = feats @ pe['w1'].T + pe['b1']
    h = _bn(h, pe['bn_g'], pe['bn_b'], pe['bn_m'], pe['bn_v'])
    h = jnp.maximum(h, 0.0)
    return h @ pe['w2'].T + pe['b2']


def kernel(epoch, hidden_states, _X, edge_index, edge_type, _node_type, _node_feature_extra, params):
    del epoch
    p = params

    loop = jnp.arange(N, dtype=edge_index.dtype)
    src = jnp.concatenate([edge_index[0], loop])
    dst = jnp.concatenate([edge_index[1], loop])
    etp = jnp.concatenate([edge_type, jnp.full((N,), N_ETYPE, edge_type.dtype)])
    ht = _node_type[src]
    tt = _node_type[dst]
    code = etp * 16 + ht * 4 + tt
    cnt = jax.ops.segment_sum(jnp.ones((E_TOT,), jnp.float32), src, num_segments=N)
    cnt_src = cnt[src]
    gidx = dst * NCODE + code
    ind = jnp.repeat(jnp.eye(HEADS, dtype=jnp.float32), DPH, axis=0)
    eye_tile = jnp.tile(jnp.eye(HEADS, dtype=jnp.float32), (NCODE, 1))

    emb_tab = _edge_table(p['edge_enc'])

    context = hidden_states[:, 0, :] @ p['proj_out_w'].T + p['proj_out_b']
    kt = context @ p['Wk_w'].T + p['Wk_b']
    v = context @ p['Wv_w'].T + p['Wv_b']

    nkey = jax.random.key(1234)
    noises = []
    for _ in range(K_LAYERS):
        nkey, sub = jax.random.split(nkey)
        noises.append(jax.random.laplace(sub, (BS, N_NODE, 1), dtype=jnp.float32)[:, :, 0])

    X = _X
    for li in range(K_LAYERS):
        g = p['gnn'][li]
        w_all = jnp.concatenate(
            [g['key_w'][:, : 2 * HID].T, g['msg_w'][:, : 2 * HID].T, g['q_w'].T], axis=1
        )
        b_all = jnp.concatenate([g['key_b'], g['msg_b'], g['q_b']])
        proj = _node_proj(X, _node_feature_extra, w_all, b_all)
        mx = proj[:, HID : 2 * HID]

        ke_tab = emb_tab @ g['key_w'][:, 2 * HID :].T
        me_tab = emb_tab @ g['msg_w'][:, 2 * HID :].T

        ke3 = ke_tab.reshape(NCODE, HEADS, 1, DPH)
        keb = (ke3 * jnp.eye(HEADS, dtype=jnp.float32)[None, :, :, None]).reshape(
            NCODE * HEADS, HID
        )
        keb_aug = jnp.concatenate([keb, eye_tile], axis=1)
        t2 = _ttab(proj[:, :HID], proj[:, 2 * HID :], keb_aug.T, ind)

        msg_e = mx[src] + me_tab[code]
        scores = t2.reshape(N * NCODE, HEADS)[gidx]
        smax = jax.ops.segment_max(scores, src, num_segments=N)
        ex = jnp.exp(scores - smax[src])
        den = jax.ops.segment_sum(ex, src, num_segments=N)
        alpha = ex / (den[src] + 1e-16) * cnt_src[:, None]
        outm = (msg_e.reshape(E_TOT, HEADS, DPH) * alpha[:, :, None]).reshape(E_TOT, HID)
        aggr = jax.ops.segment_sum(outm, dst, num_segments=N)

        bn4 = jnp.stack([g['mlp_bn_g'], g['mlp_bn_b'], g['mlp_bn_m'], g['mlp_bn_v']])
        out, mx = _mlp_stage(aggr, g['mlp_w1'].T, g['mlp_b1'], bn4, g['mlp_w2'].T, g['mlp_b2'])
        X = _xattn_stage(out, p['Wq_w'].T, p['Wq_b'], kt, v, noises[li], mx)
    return X


# speedup vs baseline: 1.1697x; 1.1697x over previous
import functools

import jax
import jax.numpy as jnp
import numpy as np
from jax.experimental import pallas as pl

BS = 50
N_NODE = 200
N = BS * N_NODE
E = 160000
HID = 200
SENT = 1024
SEQ = 100
N_ETYPE = 38
N_NTYPE = 4
HEADS = 4
DPH = HID // HEADS
K_LAYERS = 5
EP_1 = 1.0
EP_2 = 1.0
E_TOT = E + N


def _bn(x, g, b, m, v):
    return (x - m) / jnp.sqrt(v + 1e-5) * g + b


_BM = 1000


def _proj_body(x_ref, e_ref, w_ref, b_ref, o_ref):
    xx = jnp.concatenate([x_ref[...], e_ref[...]], axis=1)
    o_ref[...] = (
        jnp.dot(xx, w_ref[...], preferred_element_type=jnp.float32) + b_ref[...][None, :]
    )


def _node_proj(x, extra, w, b):
    return pl.pallas_call(
        _proj_body,
        grid=(N // _BM,),
        in_specs=[
            pl.BlockSpec((_BM, HID), lambda i: (i, 0)),
            pl.BlockSpec((_BM, HID), lambda i: (i, 0)),
            pl.BlockSpec((2 * HID, 3 * HID), lambda i: (0, 0)),
            pl.BlockSpec((3 * HID,), lambda i: (0,)),
        ],
        out_specs=pl.BlockSpec((_BM, 3 * HID), lambda i: (i, 0)),
        out_shape=jax.ShapeDtypeStruct((N, 3 * HID), jnp.float32),
    )(x, extra, w, b)


NCODE = 624
_BE = 2000
assert E_TOT % _BE == 0


def _edgemm_body(x_ref, w_ref, o_ref):
    o_ref[...] = jnp.dot(x_ref[...], w_ref[...], preferred_element_type=jnp.float32)


def _edgemm(emb_g, w):
    return pl.pallas_call(
        _edgemm_body,
        grid=(E_TOT // _BE,),
        in_specs=[
            pl.BlockSpec((_BE, HID), lambda i: (i, 0)),
            pl.BlockSpec((HID, 2 * HID), lambda i: (0, 0)),
        ],
        out_specs=pl.BlockSpec((_BE, 2 * HID), lambda i: (i, 0)),
        out_shape=jax.ShapeDtypeStruct((E_TOT, 2 * HID), jnp.float32),
    )(emb_g, w)


def _mlp_body(aggr_ref, w1_ref, b1_ref, bn_ref, w2_ref, b2_ref, o_ref, mx_ref):
    h = (
        jnp.dot(aggr_ref[...], w1_ref[...], preferred_element_type=jnp.float32)
        + b1_ref[...][None, :]
    )
    bn = bn_ref[...]
    h = _bn(h, bn[0][None, :], bn[1][None, :], bn[2][None, :], bn[3][None, :])
    h = jnp.maximum(h, 0.0)
    out = (
        jnp.dot(h, w2_ref[...], preferred_element_type=jnp.float32) + b2_ref[...][None, :]
    )
    o_ref[...] = out
    bmax = jnp.max(jnp.sum(out * out, axis=1)).reshape(1, 1)

    @pl.when(pl.program_id(0) == 0)
    def _init():
        mx_ref[...] = bmax

    @pl.when(pl.program_id(0) != 0)
    def _acc():
        mx_ref[...] = jnp.maximum(mx_ref[...], bmax)


def _mlp_stage(aggr, w1t, b1, bn4, w2t, b2):
    return pl.pallas_call(
        _mlp_body,
        grid=(N // _BM,),
        in_specs=[
            pl.BlockSpec((_BM, HID), lambda i: (i, 0)),
            pl.BlockSpec((HID, HID), lambda i: (0, 0)),
            pl.BlockSpec((HID,), lambda i: (0,)),
            pl.BlockSpec((4, HID), lambda i: (0, 0)),
            pl.BlockSpec((HID, HID), lambda i: (0, 0)),
            pl.BlockSpec((HID,), lambda i: (0,)),
        ],
        out_specs=[
            pl.BlockSpec((_BM, HID), lambda i: (i, 0)),
            pl.BlockSpec((1, 1), lambda i: (0, 0)),
        ],
        out_shape=[
            jax.ShapeDtypeStruct((N, HID), jnp.float32),
            jax.ShapeDtypeStruct((1, 1), jnp.float32),
        ],
    )(aggr, w1t, b1, bn4, w2t, b2)


def _xattn_body(out_ref, wq_ref, bq_ref, kt_ref, v_ref, noise_ref, mx_ref, o_ref):
    out = out_ref[...]
    scale = 2.0 * jnp.sqrt(mx_ref[...]) / EP_2 / (EP_1 / 4.0)
    x = out * 0.5 * (1.0 + jax.lax.erf(out / np.float32(np.sqrt(2.0))))
    q = jnp.dot(x, wq_ref[...], preferred_element_type=jnp.float32) + bq_ref[...][None, :]
    qn = jnp.sqrt(jnp.sum(q * q, axis=1, keepdims=True))
    q = q / jnp.maximum(qn, 1e-12)
    kt = kt_ref[...].reshape(1, HID)
    att = jnp.sum(q * kt, axis=1, keepdims=True) / np.float32(np.sqrt(HID))
    att = att + noise_ref[...].reshape(N_NODE, 1) * scale
    o_ref[...] = att * v_ref[...].reshape(1, HID)


def _xattn_stage(out, wqt, bq, kt, v, noise, mx):
    return pl.pallas_call(
        _xattn_body,
        grid=(BS,),
        in_specs=[
            pl.BlockSpec((N_NODE, HID), lambda i: (i, 0)),
            pl.BlockSpec((HID, HID), lambda i: (0, 0)),
            pl.BlockSpec((HID,), lambda i: (0,)),
            pl.BlockSpec((1, 1, HID), lambda i: (i, 0, 0)),
            pl.BlockSpec((1, 1, HID), lambda i: (i, 0, 0)),
            pl.BlockSpec((1, 1, N_NODE), lambda i: (i, 0, 0)),
            pl.BlockSpec((1, 1), lambda i: (0, 0)),
        ],
        out_specs=pl.BlockSpec((N_NODE, HID), lambda i: (i, 0)),
        out_shape=jax.ShapeDtypeStruct((N, HID), jnp.float32),
    )(out, wqt, bq, kt.reshape(BS, 1, HID), v.reshape(BS, 1, HID),
      noise.reshape(BS, 1, N_NODE), mx)


def _edge_table(pe):
    codes = jnp.arange(624, dtype=jnp.int32)
    et = codes // 16
    ht = (codes // 4) % 4
    tt = codes % 4
    feats = jnp.concatenate(
        [
            jax.nn.one_hot(et, N_ETYPE + 1, dtype=jnp.float32),
            jax.nn.one_hot(ht, N_NTYPE, dtype=jnp.float32),
            jax.nn.one_hot(tt, N_NTYPE, dtype=jnp.float32),
        ],
        axis=1,
    )
    h = feats @ pe['w1'].T + pe['b1']
    h = _bn(h, pe['bn_g'], pe['bn_b'], pe['bn_m'], pe['bn_v'])
    h = jnp.maximum(h, 0.0)
    return h @ pe['w2'].T + pe['b2']


def kernel(epoch, hidden_states, _X, edge_index, edge_type, _node_type, _node_feature_extra, params):
    del epoch
    p = params

    loop = jnp.arange(N, dtype=edge_index.dtype)
    src = jnp.concatenate([edge_index[0], loop])
    dst = jnp.concatenate([edge_index[1], loop])
    etp = jnp.concatenate([edge_type, jnp.full((N,), N_ETYPE, edge_type.dtype)])
    ht = _node_type[src]
    tt = _node_type[dst]
    code = etp * 16 + ht * 4 + tt
    cnt = jax.ops.segment_sum(jnp.ones((E_TOT,), jnp.float32), src, num_segments=N)
    cnt_src = cnt[src]

    emb_tab = _edge_table(p['edge_enc'])
    emb_g = emb_tab[code]

    context = hidden_states[:, 0, :] @ p['proj_out_w'].T + p['proj_out_b']
    kt = context @ p['Wk_w'].T + p['Wk_b']
    v = context @ p['Wv_w'].T + p['Wv_b']

    nkey = jax.random.key(1234)
    noises = []
    for _ in range(K_LAYERS):
        nkey, sub = jax.random.split(nkey)
        noises.append(jax.random.laplace(sub, (BS, N_NODE, 1), dtype=jnp.float32)[:, :, 0])

    X = _X
    for li in range(K_LAYERS):
        g = p['gnn'][li]
        s = np.float32(1.0 / np.sqrt(DPH))
        w_all = jnp.concatenate(
            [g['key_w'][:, : 2 * HID].T, g['q_w'].T * s, g['msg_w'][:, : 2 * HID].T], axis=1
        )
        b_all = jnp.concatenate([g['key_b'], g['q_b'] * s, g['msg_b']])
        proj = _node_proj(X, _node_feature_extra, w_all, b_all)
        kq = proj[:, : 2 * HID]
        mx = proj[:, 2 * HID :]

        w_km = jnp.concatenate(
            [g['key_w'][:, 2 * HID :].T, g['msg_w'][:, 2 * HID :].T], axis=1
        )
        kem = _edgemm(emb_g, w_km)

        gkq = kq[dst]
        key_e = gkq[:, :HID] + kem[:, :HID]
        scores = jnp.sum(
            (gkq[:, HID:] * key_e).reshape(E_TOT, HEADS, DPH), axis=2
        )
        smax = jax.ops.segment_max(scores, src, num_segments=N)
        ex = jnp.exp(scores - smax[src])
        den = jax.ops.segment_sum(ex, src, num_segments=N)
        alpha = ex / (den[src] + 1e-16) * cnt_src[:, None]
        msg_e = mx[src] + kem[:, HID:]
        outm = (msg_e.reshape(E_TOT, HEADS, DPH) * alpha[:, :, None]).reshape(E_TOT, HID)
        aggr = jax.ops.segment_sum(outm, dst, num_segments=N)

        bn4 = jnp.stack([g['mlp_bn_g'], g['mlp_bn_b'], g['mlp_bn_m'], g['mlp_bn_v']])
        out, mx = _mlp_stage(aggr, g['mlp_w1'].T, g['mlp_b1'], bn4, g['mlp_w2'].T, g['mlp_b2'])
        X = _xattn_stage(out, p['Wq_w'].T, p['Wq_b'], kt, v, noises[li], mx)
    return X


# speedup vs baseline: 1.1845x; 1.0126x over previous
import functools

import jax
import jax.numpy as jnp
import numpy as np
from jax.experimental import pallas as pl

BS = 50
N_NODE = 200
N = BS * N_NODE
E = 160000
HID = 200
SENT = 1024
SEQ = 100
N_ETYPE = 38
N_NTYPE = 4
HEADS = 4
DPH = HID // HEADS
K_LAYERS = 5
EP_1 = 1.0
EP_2 = 1.0
E_TOT = E + N


def _bn(x, g, b, m, v):
    return (x - m) / jnp.sqrt(v + 1e-5) * g + b


_BM = 1000


def _proj_body(x_ref, e_ref, w_ref, b_ref, o_ref):
    xx = jnp.concatenate([x_ref[...], e_ref[...]], axis=1)
    o_ref[...] = (
        jnp.dot(xx, w_ref[...], preferred_element_type=jnp.float32) + b_ref[...][None, :]
    )


def _node_proj(x, extra, w, b):
    return pl.pallas_call(
        _proj_body,
        grid=(N // _BM,),
        in_specs=[
            pl.BlockSpec((_BM, HID), lambda i: (i, 0)),
            pl.BlockSpec((_BM, HID), lambda i: (i, 0)),
            pl.BlockSpec((2 * HID, 3 * HID), lambda i: (0, 0)),
            pl.BlockSpec((3 * HID,), lambda i: (0,)),
        ],
        out_specs=pl.BlockSpec((_BM, 3 * HID), lambda i: (i, 0)),
        out_shape=jax.ShapeDtypeStruct((N, 3 * HID), jnp.float32),
    )(x, extra, w, b)


NCODE = 624
_BE = 2000
assert E_TOT % _BE == 0


def _edgemm_body(x_ref, w_ref, o_ref):
    o_ref[...] = jnp.dot(x_ref[...], w_ref[...], preferred_element_type=jnp.float32)


def _edgemm(emb_g, w):
    return pl.pallas_call(
        _edgemm_body,
        grid=(E_TOT // _BE,),
        in_specs=[
            pl.BlockSpec((_BE, HID), lambda i: (i, 0)),
            pl.BlockSpec((HID, 2 * HID), lambda i: (0, 0)),
        ],
        out_specs=pl.BlockSpec((_BE, 2 * HID), lambda i: (i, 0)),
        out_shape=jax.ShapeDtypeStruct((E_TOT, 2 * HID), jnp.float32),
    )(emb_g, w)


def _mlp_body(aggr_ref, w1_ref, b1_ref, bn_ref, w2_ref, b2_ref, o_ref, mx_ref):
    h = (
        jnp.dot(aggr_ref[...], w1_ref[...], preferred_element_type=jnp.float32)
        + b1_ref[...][None, :]
    )
    bn = bn_ref[...]
    h = _bn(h, bn[0][None, :], bn[1][None, :], bn[2][None, :], bn[3][None, :])
    h = jnp.maximum(h, 0.0)
    out = (
        jnp.dot(h, w2_ref[...], preferred_element_type=jnp.float32) + b2_ref[...][None, :]
    )
    o_ref[...] = out
    bmax = jnp.max(jnp.sum(out * out, axis=1)).reshape(1, 1)

    @pl.when(pl.program_id(0) == 0)
    def _init():
        mx_ref[...] = bmax

    @pl.when(pl.program_id(0) != 0)
    def _acc():
        mx_ref[...] = jnp.maximum(mx_ref[...], bmax)


def _mlp_stage(aggr, w1t, b1, bn4, w2t, b2):
    return pl.pallas_call(
        _mlp_body,
        grid=(N // _BM,),
        in_specs=[
            pl.BlockSpec((_BM, HID), lambda i: (i, 0)),
            pl.BlockSpec((HID, HID), lambda i: (0, 0)),
            pl.BlockSpec((HID,), lambda i: (0,)),
            pl.BlockSpec((4, HID), lambda i: (0, 0)),
            pl.BlockSpec((HID, HID), lambda i: (0, 0)),
            pl.BlockSpec((HID,), lambda i: (0,)),
        ],
        out_specs=[
            pl.BlockSpec((_BM, HID), lambda i: (i, 0)),
            pl.BlockSpec((1, 1), lambda i: (0, 0)),
        ],
        out_shape=[
            jax.ShapeDtypeStruct((N, HID), jnp.float32),
            jax.ShapeDtypeStruct((1, 1), jnp.float32),
        ],
    )(aggr, w1t, b1, bn4, w2t, b2)


def _xattn_body(out_ref, wq_ref, bq_ref, kt_ref, v_ref, noise_ref, mx_ref, o_ref):
    out = out_ref[...]
    scale = 2.0 * jnp.sqrt(mx_ref[...]) / EP_2 / (EP_1 / 4.0)
    x = out * 0.5 * (1.0 + jax.lax.erf(out / np.float32(np.sqrt(2.0))))
    q = jnp.dot(x, wq_ref[...], preferred_element_type=jnp.float32) + bq_ref[...][None, :]
    qn = jnp.sqrt(jnp.sum(q * q, axis=1, keepdims=True))
    q = q / jnp.maximum(qn, 1e-12)
    kt = kt_ref[...].reshape(1, HID)
    att = jnp.sum(q * kt, axis=1, keepdims=True) / np.float32(np.sqrt(HID))
    att = att + noise_ref[...].reshape(N_NODE, 1) * scale
    o_ref[...] = att * v_ref[...].reshape(1, HID)


def _xattn_stage(out, wqt, bq, kt, v, noise, mx):
    return pl.pallas_call(
        _xattn_body,
        grid=(BS,),
        in_specs=[
            pl.BlockSpec((N_NODE, HID), lambda i: (i, 0)),
            pl.BlockSpec((HID, HID), lambda i: (0, 0)),
            pl.BlockSpec((HID,), lambda i: (0,)),
            pl.BlockSpec((1, 1, HID), lambda i: (i, 0, 0)),
            pl.BlockSpec((1, 1, HID), lambda i: (i, 0, 0)),
            pl.BlockSpec((1, 1, N_NODE), lambda i: (i, 0, 0)),
            pl.BlockSpec((1, 1), lambda i: (0, 0)),
        ],
        out_specs=pl.BlockSpec((N_NODE, HID), lambda i: (i, 0)),
        out_shape=jax.ShapeDtypeStruct((N, HID), jnp.float32),
    )(out, wqt, bq, kt.reshape(BS, 1, HID), v.reshape(BS, 1, HID),
      noise.reshape(BS, 1, N_NODE), mx)


def _edge_table(pe):
    codes = jnp.arange(624, dtype=jnp.int32)
    et = codes // 16
    ht = (codes // 4) % 4
    tt = codes % 4
    feats = jnp.concatenate(
        [
            jax.nn.one_hot(et, N_ETYPE + 1, dtype=jnp.float32),
            jax.nn.one_hot(ht, N_NTYPE, dtype=jnp.float32),
            jax.nn.one_hot(tt, N_NTYPE, dtype=jnp.float32),
        ],
        axis=1,
    )
    h = feats @ pe['w1'].T + pe['b1']
    h = _bn(h, pe['bn_g'], pe['bn_b'], pe['bn_m'], pe['bn_v'])
    h = jnp.maximum(h, 0.0)
    return h @ pe['w2'].T + pe['b2']


def kernel(epoch, hidden_states, _X, edge_index, edge_type, _node_type, _node_feature_extra, params):
    del epoch
    p = params

    loop = jnp.arange(N, dtype=edge_index.dtype)
    src = jnp.concatenate([edge_index[0], loop])
    dst = jnp.concatenate([edge_index[1], loop])
    etp = jnp.concatenate([edge_type, jnp.full((N,), N_ETYPE, edge_type.dtype)])
    ht = _node_type[src]
    tt = _node_type[dst]
    code = etp * 16 + ht * 4 + tt
    ones_e = jnp.ones((E_TOT, 1), jnp.float32)
    cnt_src = None

    emb_tab = _edge_table(p['edge_enc'])
    emb_g = emb_tab[code]

    context = hidden_states[:, 0, :] @ p['proj_out_w'].T + p['proj_out_b']
    kt = context @ p['Wk_w'].T + p['Wk_b']
    v = context @ p['Wv_w'].T + p['Wv_b']

    nkey = jax.random.key(1234)
    noises = []
    for _ in range(K_LAYERS):
        nkey, sub = jax.random.split(nkey)
        noises.append(jax.random.laplace(sub, (BS, N_NODE, 1), dtype=jnp.float32)[:, :, 0])

    X = _X
    for li in range(K_LAYERS):
        g = p['gnn'][li]
        s = np.float32(1.0 / np.sqrt(DPH))
        w_all = jnp.concatenate(
            [g['key_w'][:, : 2 * HID].T, g['q_w'].T * s, g['msg_w'][:, : 2 * HID].T], axis=1
        )
        b_all = jnp.concatenate([g['key_b'], g['q_b'] * s, g['msg_b']])
        proj = _node_proj(X, _node_feature_extra, w_all, b_all)
        kq = proj[:, : 2 * HID]
        mx = proj[:, 2 * HID :]

        w_km = jnp.concatenate(
            [g['key_w'][:, 2 * HID :].T, g['msg_w'][:, 2 * HID :].T], axis=1
        )
        kem = _edgemm(emb_g, w_km)

        gkq = kq[dst]
        key_e = gkq[:, :HID] + kem[:, :HID]
        scores = jnp.sum(
            (gkq[:, HID:] * key_e).reshape(E_TOT, HEADS, DPH), axis=2
        )
        smax = jax.ops.segment_max(scores, src, num_segments=N)
        ex = jnp.exp(scores - smax[src])
        if cnt_src is None:
            den5 = jax.ops.segment_sum(
                jnp.concatenate([ex, ones_e], axis=1), src, num_segments=N
            )
            g5 = den5[src]
            den_src, cnt_src = g5[:, :HEADS], g5[:, HEADS]
        else:
            den = jax.ops.segment_sum(ex, src, num_segments=N)
            den_src = den[src]
        alpha = ex / (den_src + 1e-16) * cnt_src[:, None]
        msg_e = mx[src] + kem[:, HID:]
        outm = (msg_e.reshape(E_TOT, HEADS, DPH) * alpha[:, :, None]).reshape(E_TOT, HID)
        aggr = jax.ops.segment_sum(outm, dst, num_segments=N)

        bn4 = jnp.stack([g['mlp_bn_g'], g['mlp_bn_b'], g['mlp_bn_m'], g['mlp_bn_v']])
        out, mx = _mlp_stage(aggr, g['mlp_w1'].T, g['mlp_b1'], bn4, g['mlp_w2'].T, g['mlp_b2'])
        X = _xattn_stage(out, p['Wq_w'].T, p['Wq_b'], kt, v, noises[li], mx)
    return X
